# baseline (speedup 1.0000x reference)
import functools

import jax
import jax.numpy as jnp
from jax import lax
from jax.experimental import pallas as pl
from jax.experimental.pallas import tpu as pltpu
from jax.experimental.pallas import tpu_sc as plsc

NUM_USERS = 1000000
NUM_ITEMS = 100000
EMB_DIM = 64
BATCH = 16384

NC, NS, L = 2, 16, 16
DH = EMB_DIM // NC
SLAB = 8
NSLAB = DH // SLAB
RW = 131072
TW = 8192
NRND = 8
ACLAMP = 991744
TB2 = 999936
NT2 = NUM_USERS - TB2
CAP = 256
SCH = 2048
OUTW = 16512


def _body(user_hbm, item_hbm, uT_hbm, iflat_hbm, ub_hbm, utail_hbm,
          out_hbm,
          scan_u, scan_i, uloc_v, ilist_v, posl_v, acc_v, ival_v,
          gidx16_v, gidx2_v, row_v, tail_v, sem):
    c = lax.axis_index("c")
    s = lax.axis_index("s")
    d0 = c * DH

    pltpu.sync_copy(utail_hbm, tail_v)

    def prefill(q, _):
        sl = pl.ds(q * L, L)
        z = jnp.zeros((L,), jnp.int32)
        uloc_v[sl] = z
        ilist_v[sl] = z
        posl_v[sl] = z + BATCH
        return _

    lax.fori_loop(0, NRND * CAP // L, prefill, None)

    def scan_chunk(ch, ns):
        pltpu.sync_copy(user_hbm.at[pl.ds(ch * (SCH // 128), SCH // 128)],
                        scan_u)
        pltpu.sync_copy(item_hbm.at[pl.ds(ch * (SCH // 128), SCH // 128)],
                        scan_i)

        def group(g, ns):
            rr = g // 8
            co = (g % 8) * L
            u = scan_u[rr, pl.ds(co, L)]
            it = scan_i[rr, pl.ds(co, L)]
            r = lax.shift_right_logical(u, 17)
            tloc = jnp.bitwise_and(lax.shift_right_logical(u, 13), NS - 1)
            abase = jnp.minimum(r * RW + s * TW, ACLAMP)
            rel = u - abase
            mt = tloc == s
            posg = ch * SCH + g * L + lax.iota(jnp.int32, L)
            new_ns = []
            for q in range(NRND):
                n = ns[q]
                m = jnp.logical_and(mt, r == q)
                nn = jnp.minimum(n, CAP - L) + q * CAP
                plsc.store_compressed(uloc_v.at[pl.ds(nn, L)], rel, mask=m)
                plsc.store_compressed(ilist_v.at[pl.ds(nn, L)], it, mask=m)
                plsc.store_compressed(posl_v.at[pl.ds(nn, L)], posg, mask=m)
                pc = plsc.all_reduce_population_count(m)
                new_ns.append(jnp.minimum(n + jnp.max(pc), CAP - L))
            return tuple(new_ns)

        return lax.fori_loop(0, SCH // L, group, ns)

    lax.fori_loop(0, BATCH // SCH, scan_chunk,
                  tuple(jnp.int32(0) for _ in range(NRND)))

    core0 = jnp.where(c == 0, 1.0, 0.0).astype(jnp.float32)
    ib_off = EMB_DIM * NUM_ITEMS

    def seed_round(q, _):
        abase = jnp.minimum(q * RW + s * TW, ACLAMP)

        def mk_u(j, _):
            sl = pl.ds((j % 8) * L, L)
            gidx2_v[j // 8, sl] = uloc_v[pl.ds(q * CAP + j * L, L)] + abase
            return _

        lax.fori_loop(0, CAP // L, mk_u, None)
        cps = [pltpu.async_copy(ub_hbm.at[gidx2_v.at[j]],
                                acc_v.at[pl.ds(q * CAP + j * 128, 128)], sem)
               for j in range(CAP // 128)]
        for cp in cps:
            cp.wait()

        def mk_i(j, _):
            sl = pl.ds((j % 8) * L, L)
            gidx2_v[j // 8, sl] = ilist_v[pl.ds(q * CAP + j * L, L)] + ib_off
            return _

        lax.fori_loop(0, CAP // L, mk_i, None)
        cps = [pltpu.async_copy(iflat_hbm.at[gidx2_v.at[j]],
                                ival_v.at[pl.ds(j * 128, 128)], sem)
               for j in range(CAP // 128)]
        for cp in cps:
            cp.wait()

        def sd(j, _):
            sl = pl.ds(q * CAP + j * L, L)
            acc_v[sl] = (acc_v[sl] + ival_v[pl.ds(j * L, L)]) * core0
            return _

        lax.fori_loop(0, CAP // L, sd, None)
        return _

    lax.fori_loop(0, NRND, seed_round, None)

    def slab_round(srq, _):
        sb = srq // NRND
        q = srq % NRND
        dbase = d0 + sb * SLAB
        abase = jnp.minimum(q * RW + s * TW, ACLAMP)

        def mk_idx(j, _):
            dd = j // 16
            h = (j // 8) % 2
            sl = pl.ds((j % 8) * L, L)
            gidx16_v[j // 8, sl] = (
                ilist_v[pl.ds(q * CAP + h * 128 + (j % 8) * L, L)]
                + (dbase + dd) * NUM_ITEMS)
            return _

        lax.fori_loop(0, SLAB * CAP // L, mk_idx, None)
        cps = [pltpu.async_copy(iflat_hbm.at[gidx16_v.at[j]],
                                ival_v.at[pl.ds(j * 128, 128)], sem)
               for j in range(SLAB * CAP // 128)]
        pltpu.sync_copy(uT_hbm.at[pl.ds(dbase, SLAB), pl.ds(abase, TW)],
                        row_v)
        for cp in cps:
            cp.wait()

        def mac_d(dd, _):
            drow = jnp.full((L,), dd, jnp.int32)
            tb = jnp.full((L,), (dbase + dd) * NT2, jnp.int32)

            def mac(g, _):
                sl = pl.ds(q * CAP + g * L, L)
                rel = uloc_v[sl]
                uv = plsc.load_gather(
                    row_v, [drow, jnp.minimum(rel, TW - 1)])
                toff = rel - TW
                tv = plsc.load_gather(tail_v, [tb + jnp.maximum(toff, 0)])
                uv = jnp.where(toff >= 0, tv, uv)
                acc_v[sl] = (acc_v[sl]
                             + uv * ival_v[pl.ds(dd * CAP + g * L, L)])
                return _

            lax.fori_loop(0, CAP // L, mac, None)
            return _

        lax.fori_loop(0, SLAB, mac_d, None)
        return _

    lax.fori_loop(0, NSLAB * NRND, slab_round, None)

    off = c * OUTW

    def scat_round(q, _):
        def mk_p(j, _):
            sl = pl.ds((j % 8) * L, L)
            gidx2_v[j // 8, sl] = posl_v[pl.ds(q * CAP + j * L, L)] + off
            return _

        lax.fori_loop(0, CAP // L, mk_p, None)
        cps = [pltpu.async_copy(acc_v.at[pl.ds(q * CAP + j * 128, 128)],
                                out_hbm.at[gidx2_v.at[j]], sem)
               for j in range(CAP // 128)]
        for cp in cps:
            cp.wait()
        return _

    lax.fori_loop(0, NRND, scat_round, None)


@functools.partial(jax.jit, static_argnames=())
def kernel(user, item, user_emb, item_emb, user_bias, item_bias, global_bias):
    user2d = user.reshape(BATCH // 128, 128)
    item2d = item.reshape(BATCH // 128, 128)
    uT = user_emb.T
    iaug = jnp.concatenate(
        [item_emb.T, item_bias.reshape(1, NUM_ITEMS)], axis=0).reshape(-1)
    ub1d = user_bias.reshape(NUM_USERS) + global_bias
    utail = user_emb[TB2:].T.reshape(-1)

    run = pl.kernel(
        _body,
        out_type=jax.ShapeDtypeStruct((NC * OUTW,), jnp.float32),
        mesh=plsc.VectorSubcoreMesh(core_axis_name="c", subcore_axis_name="s",
                                    num_cores=NC, num_subcores=NS),
        scratch_types=[
            pltpu.VMEM((SCH // 128, 128), jnp.int32),
            pltpu.VMEM((SCH // 128, 128), jnp.int32),
            pltpu.VMEM((NRND * CAP,), jnp.int32),
            pltpu.VMEM((NRND * CAP,), jnp.int32),
            pltpu.VMEM((NRND * CAP,), jnp.int32),
            pltpu.VMEM((NRND * CAP,), jnp.float32),
            pltpu.VMEM((SLAB * CAP,), jnp.float32),
            pltpu.VMEM((SLAB * CAP // 128, 128), jnp.int32),
            pltpu.VMEM((CAP // 128, 128), jnp.int32),
            pltpu.VMEM((SLAB, TW), jnp.float32),
            pltpu.VMEM((NT2 * EMB_DIM,), jnp.float32),
            pltpu.SemaphoreType.DMA,
        ],
        compiler_params=pltpu.CompilerParams(needs_layout_passes=False,
                                             use_tc_tiling_on_sc=True),
    )
    parts = run(user2d, item2d, uT, iaug, ub1d, utail)
    return parts[:BATCH] + parts[OUTW:OUTW + BATCH]
